# baseline (speedup 1.0000x reference)
import functools

import jax
import jax.numpy as jnp
from jax import lax
from jax.experimental import pallas as pl
from jax.experimental.pallas import tpu as pltpu
from jax.experimental.pallas import tpu_sc as plsc

_B = 16384
_D = 64
_L = 16

_info = plsc.get_sparse_core_info()
_NC, _NS = _info.num_cores, _info.num_subcores
_NW = _NC * _NS
_BPW = _B // _NW


def _gmf_body(users_hbm, items_hbm, ut_hbm, it_hbm, out_hbm,
              uidx_v, iidx_v, urows_v, irows_v, sem_u, sem_i):
    wid = lax.axis_index("s") * _NC + lax.axis_index("c")
    base = wid * _BPW

    pltpu.sync_copy(users_hbm.at[pl.ds(base, _BPW)], uidx_v)
    pltpu.sync_copy(items_hbm.at[pl.ds(base, _BPW)], iidx_v)

    cu = pltpu.async_copy(ut_hbm.at[uidx_v], urows_v, sem_u)
    ci = pltpu.async_copy(it_hbm.at[iidx_v], irows_v, sem_i)
    cu.wait()
    ci.wait()

    def mul_row(r, carry):
        for c in range(_D // _L):
            sl = pl.ds(c * _L, _L)
            urows_v[r, sl] = urows_v[r, sl] * irows_v[r, sl]
        return carry

    lax.fori_loop(0, _BPW, mul_row, 0)

    pltpu.sync_copy(urows_v, out_hbm.at[pl.ds(base, _BPW)])


@jax.jit
def _gmf(users, items, user_table, item_table):
    mesh = plsc.VectorSubcoreMesh(core_axis_name="c", subcore_axis_name="s")
    run = pl.kernel(
        _gmf_body,
        out_type=jax.ShapeDtypeStruct((_B, _D), jnp.float32),
        mesh=mesh,
        scratch_types=[
            pltpu.VMEM((_BPW,), jnp.int32),
            pltpu.VMEM((_BPW,), jnp.int32),
            pltpu.VMEM((_BPW, _D), jnp.float32),
            pltpu.VMEM((_BPW, _D), jnp.float32),
            pltpu.SemaphoreType.DMA,
            pltpu.SemaphoreType.DMA,
        ],
        compiler_params=pltpu.CompilerParams(use_tc_tiling_on_sc=False),
    )
    return run(users, items, user_table, item_table)


def kernel(users, items, user_table, item_table):
    return _gmf(users.astype(jnp.int32), items.astype(jnp.int32),
                user_table, item_table)


# speedup vs baseline: 1.5504x; 1.5504x over previous
import functools

import jax
import jax.numpy as jnp
from jax import lax
from jax.experimental import pallas as pl
from jax.experimental.pallas import tpu as pltpu
from jax.experimental.pallas import tpu_sc as plsc

_B = 16384
_D = 64
_L = 16

_info = plsc.get_sparse_core_info()
_NC, _NS = _info.num_cores, _info.num_subcores
_NW = _NC * _NS
_BPW = _B // _NW
_CH = 128
_NCH = _BPW // _CH


def _gmf_body(users_hbm, items_hbm, ut_hbm, it_hbm, out_hbm,
              uidx_v, iidx_v,
              urows_v, irows_v, prod_v, sem_u, sem_i):
    wid = lax.axis_index("s") * _NC + lax.axis_index("c")
    base = wid * _BPW

    pltpu.sync_copy(users_hbm.at[pl.ds(base, _BPW)], uidx_v.at[pl.ds(0, _BPW)])
    pltpu.sync_copy(items_hbm.at[pl.ds(base, _BPW)], iidx_v.at[pl.ds(0, _BPW)])
    def do_chunk(ch, carry):
        def fetch_row(r, c2):
            u = uidx_v[pl.ds(ch * _CH + r, _L)][0]
            i = iidx_v[pl.ds(ch * _CH + r, _L)][0]
            pltpu.async_copy(ut_hbm.at[u], urows_v.at[r], sem_u)
            pltpu.async_copy(it_hbm.at[i], irows_v.at[r], sem_i)
            return c2

        lax.fori_loop(0, _CH, fetch_row, 0)

        pltpu.make_async_copy(
            ut_hbm.at[pl.ds(0, _CH), :], urows_v, sem_u).wait()
        pltpu.make_async_copy(
            it_hbm.at[pl.ds(0, _CH), :], irows_v, sem_i).wait()

        def mul_row(r, c2):
            for g in range(_D // _L):
                sl = pl.ds(g * _L, _L)
                prod_v[pl.ds((ch * _CH + r) * _D + g * _L, _L)] = (
                    urows_v[r, sl] * irows_v[r, sl])
            return c2

        lax.fori_loop(0, _CH, mul_row, 0)
        return carry

    lax.fori_loop(0, _NCH, do_chunk, 0)

    pltpu.sync_copy(prod_v, out_hbm.at[pl.ds(base * _D, _BPW * _D)])


@jax.jit
def _gmf(users, items, user_table, item_table):
    mesh = plsc.VectorSubcoreMesh(core_axis_name="c", subcore_axis_name="s")
    run = pl.kernel(
        _gmf_body,
        out_type=jax.ShapeDtypeStruct((_B * _D,), jnp.float32),
        mesh=mesh,
        scratch_types=[
            pltpu.VMEM((_BPW + _L,), jnp.int32),
            pltpu.VMEM((_BPW + _L,), jnp.int32),
            pltpu.VMEM((_CH, _D), jnp.float32),
            pltpu.VMEM((_CH, _D), jnp.float32),
            pltpu.VMEM((_BPW * _D,), jnp.float32),
            pltpu.SemaphoreType.DMA,
            pltpu.SemaphoreType.DMA,
        ],
        compiler_params=pltpu.CompilerParams(disable_bounds_checks=True),
    )
    return run(users, items, user_table, item_table).reshape(_B, _D)


def kernel(users, items, user_table, item_table):
    return _gmf(users.astype(jnp.int32), items.astype(jnp.int32),
                user_table, item_table)
